# baseline (speedup 1.0000x reference)
import functools

import jax
import jax.numpy as jnp
from jax import lax
from jax.experimental import pallas as pl
from jax.experimental.pallas import tpu as pltpu
from jax.experimental.pallas import tpu_sc as plsc

NC = 2
NS = 16
L = 16

BB = 4
NP = 4096
NW_PER_B = 8
RPW = NP // NW_PER_B
RG = 8
NG = RPW // RG
NJ = NP // L
NWORK = NC * NS


def _chamfer_body(x1_hbm, x2_hbm, out1_hbm, out2_hbm,
                  x1x, x1y, x1z, a1,
                  x2x, x2y, x2z, e2,
                  umin, comb, iob, part):
    c = lax.axis_index("c")
    s = lax.axis_index("s")
    b = c * 2 + s // NW_PER_B
    bl = s // NW_PER_B
    k = s % NW_PER_B
    wid = c * NS + s
    rbase = k * RPW

    b2 = b * 3 * NP
    pltpu.sync_copy(x2_hbm.at[pl.ds(b2, NP)], x2x)
    pltpu.sync_copy(x2_hbm.at[pl.ds(b2 + NP, NP)], x2y)
    pltpu.sync_copy(x2_hbm.at[pl.ds(b2 + 2 * NP, NP)], x2z)
    pltpu.sync_copy(x1_hbm.at[pl.ds(b2 + rbase, RPW)], x1x)
    pltpu.sync_copy(x1_hbm.at[pl.ds(b2 + NP + rbase, RPW)], x1y)
    pltpu.sync_copy(x1_hbm.at[pl.ds(b2 + 2 * NP + rbase, RPW)], x1z)

    inf16 = jnp.full((L,), jnp.inf, jnp.float32)

    def bf16_round(v):
        u = lax.bitcast_convert_type(v, jnp.uint32)
        u = u + jnp.uint32(0x7FFF) + ((u >> 16) & jnp.uint32(1))
        u = u & jnp.uint32(0xFFFF0000)
        return lax.bitcast_convert_type(u, jnp.float32)

    def pre2(j, _):
        sl = pl.ds(j * L, L)
        xx = x2x[sl]
        yy = x2y[sl]
        zz = x2z[sl]
        e2[sl] = 0.5 * (xx * xx + yy * yy + zz * zz)
        x2x[sl] = bf16_round(xx)
        x2y[sl] = bf16_round(yy)
        x2z[sl] = bf16_round(zz)
        umin[sl] = inf16
        return 0
    lax.fori_loop(0, NJ, pre2, 0)

    def pre1(i, _):
        sl = pl.ds(i * L, L)
        xx = x1x[sl]
        yy = x1y[sl]
        zz = x1z[sl]
        a1[sl] = 0.5 * (xx * xx + yy * yy + zz * zz)
        x1x[sl] = -bf16_round(xx)
        x1y[sl] = -bf16_round(yy)
        x1z[sl] = -bf16_round(zz)
        return 0
    lax.fori_loop(0, RPW // L, pre1, 0)

    def group_body(g, s1):
        gsl = pl.ds(g * L, L)
        vx = x1x[gsl]
        vy = x1y[gsl]
        vz = x1z[gsl]
        va = a1[gsl]
        for half in range(L // RG):
            nbx = []
            nby = []
            nbz = []
            av = []
            for r in range(RG):
                i = half * RG + r
                nbx.append(jnp.full((L,), vx[i]))
                nby.append(jnp.full((L,), vy[i]))
                nbz.append(jnp.full((L,), vz[i]))
                av.append(jnp.full((L,), va[i]))

            def col_body(j, tmins):
                sl = pl.ds(j * L, L)
                xx = x2x[sl]
                yy = x2y[sl]
                zz = x2z[sl]
                e = e2[sl]
                new_t = []
                ucand = None
                for r in range(RG):
                    h = av[r] + e
                    h = h + xx * nbx[r]
                    h = h + yy * nby[r]
                    h = h + zz * nbz[r]
                    new_t.append(jnp.minimum(tmins[r], h))
                    ucand = h if ucand is None else jnp.minimum(ucand, h)
                umin[sl] = jnp.minimum(umin[sl], ucand)
                return tuple(new_t)

            tmins = lax.fori_loop(0, NJ, col_body, (inf16,) * RG)
            for r in range(RG):
                m = jnp.min(tmins[r])
                s1 = s1 + jnp.maximum(m + m, 0.0)
        return s1

    s1 = lax.fori_loop(0, RPW // L, group_body, jnp.float32(0.0))

    lane = lax.iota(jnp.int32, L)
    iob[:] = jnp.where(lane == 0, s1, 0.0)
    pltpu.sync_copy(iob, out1_hbm.at[pl.ds(wid * L, L)])

    pltpu.sync_copy(umin, part.at[pl.ds((bl * NW_PER_B + k) * NP, NP)])
    plsc.subcore_barrier()
    cbase = k * RPW
    for i in range(NW_PER_B):
        pltpu.sync_copy(
            part.at[pl.ds((bl * NW_PER_B + i) * NP + cbase, RPW)],
            comb.at[i],
        )

    def col_sum(j, s2):
        sl = pl.ds(j * L, L)
        u = comb[0, sl]
        for i in range(1, NW_PER_B):
            u = jnp.minimum(u, comb[i, sl])
        return s2 + jnp.maximum(u + u, 0.0)

    s2v = lax.fori_loop(0, RPW // L, col_sum, jnp.zeros((L,), jnp.float32))
    iob[:] = s2v
    pltpu.sync_copy(iob, out2_hbm.at[pl.ds(wid * L, L)])


@jax.jit
def _chamfer_sc(x1t, x2t):
    f32 = jnp.float32
    run = pl.kernel(
        _chamfer_body,
        out_type=[
            jax.ShapeDtypeStruct((NWORK * L,), f32),
            jax.ShapeDtypeStruct((NWORK * L,), f32),
        ],
        mesh=plsc.VectorSubcoreMesh(
            core_axis_name="c", subcore_axis_name="s",
            num_cores=NC, num_subcores=NS,
        ),
        compiler_params=pltpu.CompilerParams(needs_layout_passes=False),
        scratch_types=[
            pltpu.VMEM((RPW,), f32),
            pltpu.VMEM((RPW,), f32),
            pltpu.VMEM((RPW,), f32),
            pltpu.VMEM((RPW,), f32),
            pltpu.VMEM((NP,), f32),
            pltpu.VMEM((NP,), f32),
            pltpu.VMEM((NP,), f32),
            pltpu.VMEM((NP,), f32),
            pltpu.VMEM((NP,), f32),
            pltpu.VMEM((NW_PER_B, RPW), f32),
            pltpu.VMEM((L,), f32),
            pltpu.VMEM_SHARED((2 * NW_PER_B * NP,), f32),
        ],
    )
    return run(x1t, x2t)


def kernel(xyz1, xyz2):
    x1t = jnp.transpose(xyz1, (0, 2, 1)).reshape(-1)
    x2t = jnp.transpose(xyz2, (0, 2, 1)).reshape(-1)
    o1, o2 = _chamfer_sc(x1t, x2t)
    return jnp.sum(o1) / (BB * NP) + jnp.sum(o2) / (BB * NP)
